# baseline (speedup 1.0000x reference)
import jax
import jax.numpy as jnp
from jax.experimental import pallas as pl

_B = 8
_N = 5000
_TOPK = 200
_IOU_T = 0.5
_SCORE_T = 0.01


def _decode_body(xt_ref, at_ref, x1_ref, y1_ref, x2_ref, y2_ref):
    d = xt_ref[0]
    d_x = d[0:1, :]
    d_y = d[1:2, :]
    d_w = d[2:3, :]
    d_h = d[3:4, :]
    a_x = at_ref[0:1, :]
    a_y = at_ref[1:2, :]
    a_w = at_ref[2:3, :]
    a_h = at_ref[3:4, :]
    cx = d_x * a_w / 10.0 + a_x
    cy = d_y * a_h / 10.0 + a_y
    w = jnp.exp(d_w / 5.0) * a_w
    h = jnp.exp(d_h / 5.0) * a_h
    x1_ref[0] = cx - w / 2.0
    y1_ref[0] = cy - h / 2.0
    x2_ref[0] = cx + w / 2.0
    y2_ref[0] = cy + h / 2.0


def _nms_body(f_ref, out_ref):
    sa0 = f_ref[0]
    x1 = f_ref[1]
    y1 = f_ref[2]
    x2 = f_ref[3]
    y2 = f_ref[4]
    lane = jax.lax.broadcasted_iota(jnp.int32, (_B, _N), 1)
    area = (x2 - x1) * (y2 - y1)
    k = jax.lax.broadcasted_iota(jnp.int32, (_B, 8), 1)

    def body(i, sa):
        m = jnp.max(sa, axis=1, keepdims=True)
        found = m >= _SCORE_T
        idx = jnp.min(jnp.where(sa == m, lane, 1 << 30), axis=1, keepdims=True)
        sel = lane == idx

        def pick(v):
            return jnp.sum(jnp.where(sel, v, 0.0), axis=1, keepdims=True)

        bx1 = pick(x1)
        by1 = pick(y1)
        bx2 = pick(x2)
        by2 = pick(y2)
        a1 = (bx2 - bx1) * (by2 - by1)
        xl = jnp.maximum(bx1, x1)
        xr = jnp.minimum(bx2, x2)
        yt = jnp.maximum(by1, y1)
        yb = jnp.minimum(by2, y2)
        common = jnp.clip(xr - xl, 0.0, 1.0) * jnp.clip(yb - yt, 0.0, 1.0)
        denom = a1 + area - common
        t = common * 67108864.0 - denom * 33554432.0
        sa = jnp.where(t >= 0.0 - denom, -2.0, sa)

        vals = jnp.where(k == 0, idx.astype(jnp.float32), 0.0)
        vals = jnp.where(k == 1, m, vals)
        vals = jnp.where(k == 2, bx1, vals)
        vals = jnp.where(k == 3, by1, vals)
        vals = jnp.where(k == 4, bx2, vals)
        vals = jnp.where(k == 5, by2, vals)
        vals = jnp.where(found, vals, 0.0)
        out_ref[pl.ds(i, 1), :, :] = vals[None, :, :]
        return sa

    jax.lax.fori_loop(0, _TOPK, body, sa0, unroll=2)


def kernel(x, anchor):
    cp = jax.nn.softmax(x[:, :, 4:], axis=2)
    s = jnp.max(cp[:, :, 1:], axis=2)
    c = jnp.argmax(cp[:, :, 1:], axis=2).astype(jnp.float32)

    xt = jnp.transpose(x[:, :, :4], (0, 2, 1))
    at = anchor.T

    row = jax.ShapeDtypeStruct((_B, 1, _N), jnp.float32)
    boxes = pl.pallas_call(
        _decode_body,
        grid=(_B,),
        in_specs=[
            pl.BlockSpec((1, 4, _N), lambda b: (b, 0, 0)),
            pl.BlockSpec((4, _N), lambda b: (0, 0)),
        ],
        out_specs=[pl.BlockSpec((1, 1, _N), lambda b: (b, 0, 0))] * 4,
        out_shape=[row] * 4,
    )(xt, at)

    fields = jnp.stack(
        [s] + [b.reshape(_B, _N) for b in boxes], axis=0
    )

    out = pl.pallas_call(
        _nms_body,
        out_shape=jax.ShapeDtypeStruct((_TOPK, _B, 8), jnp.float32),
    )(fields)
    out = jnp.transpose(out, (1, 0, 2))
    scores = out[:, :, 1]
    cls = jnp.take_along_axis(c, out[:, :, 0].astype(jnp.int32), axis=1)
    cls = jnp.where(scores >= _SCORE_T, cls, 0.0)
    return jnp.concatenate([cls[:, :, None], out[:, :, 1:6]], axis=2)


# speedup vs baseline: 1.1430x; 1.1430x over previous
import jax
import jax.numpy as jnp
from jax.experimental import pallas as pl

_B = 8
_N = 5000
_TOPK = 200
_IOU_T = 0.5
_SCORE_T = 0.01


def _decode_body(xt_ref, at_ref, x1_ref, y1_ref, x2_ref, y2_ref):
    d = xt_ref[0]
    d_x = d[0:1, :]
    d_y = d[1:2, :]
    d_w = d[2:3, :]
    d_h = d[3:4, :]
    a_x = at_ref[0:1, :]
    a_y = at_ref[1:2, :]
    a_w = at_ref[2:3, :]
    a_h = at_ref[3:4, :]
    cx = d_x * a_w / 10.0 + a_x
    cy = d_y * a_h / 10.0 + a_y
    w = jnp.exp(d_w / 5.0) * a_w
    h = jnp.exp(d_h / 5.0) * a_h
    x1_ref[0] = cx - w / 2.0
    y1_ref[0] = cy - h / 2.0
    x2_ref[0] = cx + w / 2.0
    y2_ref[0] = cy + h / 2.0


def _nms_body(f_ref, out_ref):
    sa0 = f_ref[0]
    cv = f_ref[1]
    x1 = f_ref[2]
    y1 = f_ref[3]
    x2 = f_ref[4]
    y2 = f_ref[5]
    lane = jax.lax.broadcasted_iota(jnp.int32, (_B, _N), 1)
    area = (x2 - x1) * (y2 - y1)
    k = jax.lax.broadcasted_iota(jnp.int32, (_B, 8), 1)

    def body(i, sa):
        m = jnp.max(sa, axis=1, keepdims=True)
        found = m >= _SCORE_T
        idx = jnp.min(jnp.where(sa == m, lane, 1 << 30), axis=1, keepdims=True)
        sel = lane == idx

        def pick(v):
            return jnp.sum(jnp.where(sel, v, 0.0), axis=1, keepdims=True)

        bx1 = pick(x1)
        by1 = pick(y1)
        bx2 = pick(x2)
        by2 = pick(y2)
        bc = pick(cv)
        a1 = (bx2 - bx1) * (by2 - by1)
        xl = jnp.maximum(bx1, x1)
        xr = jnp.minimum(bx2, x2)
        yt = jnp.maximum(by1, y1)
        yb = jnp.minimum(by2, y2)
        common = jnp.clip(xr - xl, 0.0, 1.0) * jnp.clip(yb - yt, 0.0, 1.0)
        denom = a1 + area - common
        t = common * 67108864.0 - denom * 33554432.0
        sa = jnp.where(t >= 0.0 - denom, -2.0, sa)

        vals = jnp.where(k == 0, bc, 0.0)
        vals = jnp.where(k == 1, m, vals)
        vals = jnp.where(k == 2, bx1, vals)
        vals = jnp.where(k == 3, by1, vals)
        vals = jnp.where(k == 4, bx2, vals)
        vals = jnp.where(k == 5, by2, vals)
        vals = jnp.where(found, vals, 0.0)
        out_ref[pl.ds(i, 1), :, :] = vals[None, :, :]
        return sa

    jax.lax.fori_loop(0, _TOPK, body, sa0, unroll=2)


def kernel(x, anchor):
    cp = jax.nn.softmax(x[:, :, 4:], axis=2)
    s = jnp.max(cp[:, :, 1:], axis=2)
    c = jnp.argmax(cp[:, :, 1:], axis=2).astype(jnp.float32)

    xt = jnp.transpose(x[:, :, :4], (0, 2, 1))
    at = anchor.T

    row = jax.ShapeDtypeStruct((_B, 1, _N), jnp.float32)
    boxes = pl.pallas_call(
        _decode_body,
        grid=(_B,),
        in_specs=[
            pl.BlockSpec((1, 4, _N), lambda b: (b, 0, 0)),
            pl.BlockSpec((4, _N), lambda b: (0, 0)),
        ],
        out_specs=[pl.BlockSpec((1, 1, _N), lambda b: (b, 0, 0))] * 4,
        out_shape=[row] * 4,
    )(xt, at)

    fields = jnp.stack(
        [s, c] + [b.reshape(_B, _N) for b in boxes], axis=0
    )

    out = pl.pallas_call(
        _nms_body,
        out_shape=jax.ShapeDtypeStruct((_TOPK, _B, 8), jnp.float32),
    )(fields)
    return jnp.transpose(out, (1, 0, 2))[:, :, :6]


# speedup vs baseline: 1.1884x; 1.0397x over previous
import jax
import jax.numpy as jnp
from jax.experimental import pallas as pl

_B = 8
_N = 5000
_TOPK = 200
_IOU_T = 0.5
_SCORE_T = 0.01


def _decode_body(xt_ref, at_ref, x1_ref, y1_ref, x2_ref, y2_ref):
    d = xt_ref[0]
    d_x = d[0:1, :]
    d_y = d[1:2, :]
    d_w = d[2:3, :]
    d_h = d[3:4, :]
    a_x = at_ref[0:1, :]
    a_y = at_ref[1:2, :]
    a_w = at_ref[2:3, :]
    a_h = at_ref[3:4, :]
    cx = d_x * a_w / 10.0 + a_x
    cy = d_y * a_h / 10.0 + a_y
    w = jnp.exp(d_w / 5.0) * a_w
    h = jnp.exp(d_h / 5.0) * a_h
    x1_ref[0] = cx - w / 2.0
    y1_ref[0] = cy - h / 2.0
    x2_ref[0] = cx + w / 2.0
    y2_ref[0] = cy + h / 2.0


def _nms_body(f_ref, out_ref):
    sa0 = f_ref[0]
    cv = f_ref[1]
    x1 = f_ref[2]
    y1 = f_ref[3]
    x2 = f_ref[4]
    y2 = f_ref[5]
    lane = jax.lax.broadcasted_iota(jnp.int32, (_B, _N), 1)
    area = (x2 - x1) * (y2 - y1)
    k = jax.lax.broadcasted_iota(jnp.int32, (_B, 8), 1)

    def body(i, sa):
        m = jnp.max(sa, axis=1, keepdims=True)
        found = m >= _SCORE_T
        idx = jnp.min(jnp.where(sa == m, lane, 1 << 30), axis=1, keepdims=True)
        sel = lane == idx

        def pick(v):
            return jnp.sum(jnp.where(sel, v, 0.0), axis=1, keepdims=True)

        bx1 = pick(x1)
        by1 = pick(y1)
        bx2 = pick(x2)
        by2 = pick(y2)
        bc = pick(cv)
        a1 = (bx2 - bx1) * (by2 - by1)
        xl = jnp.maximum(bx1, x1)
        xr = jnp.minimum(bx2, x2)
        yt = jnp.maximum(by1, y1)
        yb = jnp.minimum(by2, y2)
        common = jnp.clip(xr - xl, 0.0, 1.0) * jnp.clip(yb - yt, 0.0, 1.0)
        iou = common / (a1 + area - common)
        sa = jnp.where(iou >= _IOU_T, -2.0, sa)

        vals = jnp.where(k == 0, bc, 0.0)
        vals = jnp.where(k == 1, m, vals)
        vals = jnp.where(k == 2, bx1, vals)
        vals = jnp.where(k == 3, by1, vals)
        vals = jnp.where(k == 4, bx2, vals)
        vals = jnp.where(k == 5, by2, vals)
        vals = jnp.where(found, vals, 0.0)
        out_ref[pl.ds(i, 1), :, :] = vals[None, :, :]
        return sa

    jax.lax.fori_loop(0, _TOPK, body, sa0, unroll=2)


def kernel(x, anchor):
    cp = jax.nn.softmax(x[:, :, 4:], axis=2)
    s = jnp.max(cp[:, :, 1:], axis=2)
    c = jnp.argmax(cp[:, :, 1:], axis=2).astype(jnp.float32)

    xt = jnp.transpose(x[:, :, :4], (0, 2, 1))
    at = anchor.T

    row = jax.ShapeDtypeStruct((_B, 1, _N), jnp.float32)
    boxes = pl.pallas_call(
        _decode_body,
        grid=(_B,),
        in_specs=[
            pl.BlockSpec((1, 4, _N), lambda b: (b, 0, 0)),
            pl.BlockSpec((4, _N), lambda b: (0, 0)),
        ],
        out_specs=[pl.BlockSpec((1, 1, _N), lambda b: (b, 0, 0))] * 4,
        out_shape=[row] * 4,
    )(xt, at)

    fields = jnp.stack(
        [s, c] + [b.reshape(_B, _N) for b in boxes], axis=0
    )

    out = pl.pallas_call(
        _nms_body,
        out_shape=jax.ShapeDtypeStruct((_TOPK, _B, 8), jnp.float32),
    )(fields)
    return jnp.transpose(out, (1, 0, 2))[:, :, :6]


# speedup vs baseline: 1.2613x; 1.0614x over previous
import jax
import jax.numpy as jnp
from jax.experimental import pallas as pl

_B = 8
_N = 5000
_TOPK = 200
_IOU_T = 0.5
_SCORE_T = 0.01


def _nms_body(s_ref, c_ref, dx_ref, dy_ref, dw_ref, dh_ref,
              ax_ref, ay_ref, aw_ref, ah_ref, out_ref):
    sa0 = s_ref[...]
    cv = c_ref[...]
    a_x = ax_ref[...]
    a_y = ay_ref[...]
    a_w = aw_ref[...]
    a_h = ah_ref[...]
    cx = dx_ref[...] * a_w / 10.0 + a_x
    cy = dy_ref[...] * a_h / 10.0 + a_y
    w = jnp.exp(dw_ref[...] / 5.0) * a_w
    h = jnp.exp(dh_ref[...] / 5.0) * a_h
    x1 = cx - w / 2.0
    y1 = cy - h / 2.0
    x2 = cx + w / 2.0
    y2 = cy + h / 2.0
    lane = jax.lax.broadcasted_iota(jnp.int32, (_B, _N), 1)
    area = (x2 - x1) * (y2 - y1)
    k = jax.lax.broadcasted_iota(jnp.int32, (_B, 8), 1)

    def body(i, sa):
        m = jnp.max(sa, axis=1, keepdims=True)
        found = m >= _SCORE_T
        idx = jnp.min(jnp.where(sa == m, lane, 1 << 30), axis=1, keepdims=True)
        sel = lane == idx

        def pick(v):
            return jnp.sum(jnp.where(sel, v, 0.0), axis=1, keepdims=True)

        bx1 = pick(x1)
        by1 = pick(y1)
        bx2 = pick(x2)
        by2 = pick(y2)
        bc = pick(cv)
        a1 = (bx2 - bx1) * (by2 - by1)
        xl = jnp.maximum(bx1, x1)
        xr = jnp.minimum(bx2, x2)
        yt = jnp.maximum(by1, y1)
        yb = jnp.minimum(by2, y2)
        common = jnp.clip(xr - xl, 0.0, 1.0) * jnp.clip(yb - yt, 0.0, 1.0)
        iou = common / (a1 + area - common)
        sa = jnp.where(iou >= _IOU_T, -2.0, sa)

        vals = jnp.where(k == 0, bc, 0.0)
        vals = jnp.where(k == 1, m, vals)
        vals = jnp.where(k == 2, bx1, vals)
        vals = jnp.where(k == 3, by1, vals)
        vals = jnp.where(k == 4, bx2, vals)
        vals = jnp.where(k == 5, by2, vals)
        vals = jnp.where(found, vals, 0.0)
        out_ref[pl.ds(i, 1), :, :] = vals[None, :, :]
        return sa

    jax.lax.fori_loop(0, _TOPK, body, sa0, unroll=2)


def kernel(x, anchor):
    cp = jax.nn.softmax(x[:, :, 4:], axis=2)
    s = jnp.max(cp[:, :, 1:], axis=2)
    c = jnp.argmax(cp[:, :, 1:], axis=2).astype(jnp.float32)

    deltas = [x[:, :, i] for i in range(4)]
    anchors = [anchor[None, :, i] for i in range(4)]

    out = pl.pallas_call(
        _nms_body,
        out_shape=jax.ShapeDtypeStruct((_TOPK, _B, 8), jnp.float32),
    )(s, c, *deltas, *anchors)
    return jnp.transpose(out, (1, 0, 2))[:, :, :6]
